# baseline (speedup 1.0000x reference)
import functools

import jax
import jax.numpy as jnp
from jax import lax
from jax.experimental import pallas as pl
from jax.experimental.pallas import tpu as pltpu
from jax.experimental.pallas import tpu_sc as plsc

BS, CH, L, H, W = 2, 256, 8, 50, 50
R = 64
PH, PW = 7, 7
SCALE = 0.0625

D = CH * L
HD = D // 2
NPIX = BS * H * W
NBINS = R * PH * PW
NB = PH * PW
NW = 32
RPW = R // NW
BPW = NBINS // NW
K = 16
LANES = 16


def _bin_geometry(rois):
    b = jnp.clip(jnp.round(rois[:, 0]).astype(jnp.int32), 0, BS - 1)
    rsw = jnp.round(rois[:, 1] * SCALE).astype(jnp.int32)
    rsh = jnp.round(rois[:, 2] * SCALE).astype(jnp.int32)
    rew = jnp.round(rois[:, 3] * SCALE).astype(jnp.int32)
    reh = jnp.round(rois[:, 4] * SCALE).astype(jnp.int32)
    roi_w = jnp.maximum(rew - rsw + 1, 1)
    roi_h = jnp.maximum(reh - rsh + 1, 1)
    p = jnp.arange(PH, dtype=jnp.int32)
    hs = jnp.clip(p[None] * roi_h[:, None] // PH + rsh[:, None], 0, H)
    he = jnp.clip(((p[None] + 1) * roi_h[:, None] + PH - 1) // PH + rsh[:, None], 0, H)
    ws = jnp.clip(p[None] * roi_w[:, None] // PW + rsw[:, None], 0, W)
    we = jnp.clip(((p[None] + 1) * roi_w[:, None] + PW - 1) // PW + rsw[:, None], 0, W)
    valid = (he[:, :, None] > hs[:, :, None]) & (we[:, None, :] > ws[:, None, :])
    d4 = jnp.arange(4, dtype=jnp.int32)
    hh = jnp.clip(jnp.minimum(hs[:, :, None] + d4, he[:, :, None] - 1), 0, H - 1)
    ww = jnp.clip(jnp.minimum(ws[:, :, None] + d4, we[:, :, None] - 1), 0, W - 1)
    pid = (b[:, None, None, None, None] * (H * W)
           + hh[:, :, None, :, None] * W
           + ww[:, None, :, None, :])
    idx = pid.reshape(NBINS, K).astype(jnp.int32)
    vmul = jnp.broadcast_to(
        valid.reshape(NBINS, 1).astype(jnp.float32), (NBINS, K))
    return idx, vmul


@functools.cache
def _make_sc_pool():
    mesh = plsc.VectorSubcoreMesh(core_axis_name="c", subcore_axis_name="s")

    @functools.partial(
        pl.kernel,
        out_type=jax.ShapeDtypeStruct((R * 2, 1, HD * NB), jnp.float32),
        mesh=mesh,
        compiler_params=pltpu.CompilerParams(needs_layout_passes=False),
        scratch_types=[
            pltpu.VMEM((BPW, K), jnp.int32),
            pltpu.VMEM((BPW, K), jnp.float32),
            pltpu.VMEM((K, HD), jnp.float32),
            pltpu.VMEM((HD * NB,), jnp.float32),
            pltpu.SemaphoreType.DMA,
        ],
    )
    def _sc_pool(table_hbm, idx_hbm, vmul_hbm, out_hbm,
                 idx_v, vmul_v, rows_v, obuf_v, gsem):
        wid = lax.axis_index("s") * 2 + lax.axis_index("c")
        pltpu.sync_copy(idx_hbm.at[wid], idx_v)
        pltpu.sync_copy(vmul_hbm.at[wid], vmul_v)
        lane = lax.iota(jnp.int32, LANES)

        for rl in range(RPW):
            roi = wid * RPW + rl
            for half in range(2):

                @pl.loop(0, NB)
                def _bin_loop(j):
                    i = rl * NB + j
                    iv = idx_v[i, :] * 2 + half
                    pltpu.async_copy(table_hbm.at[iv], rows_v, gsem).wait()
                    fvec = vmul_v[i, :]

                    @pl.loop(0, HD // LANES)
                    def _d_loop(d):
                        sl = pl.ds(d * LANES, LANES)
                        acc = rows_v[0, sl]
                        for k in range(1, K):
                            acc = jnp.maximum(acc, rows_v[k, sl])
                        fidx = (d * LANES + lane) * NB + j
                        plsc.store_scatter(obuf_v, [fidx], acc * fvec)

                pltpu.sync_copy(obuf_v, out_hbm.at[roi * 2 + half, 0])

    return _sc_pool


def kernel(input, rois):
    table = jnp.transpose(input.reshape(BS, D, H * W), (0, 2, 1)).reshape(NPIX * 2, HD)
    idx, vmul = _bin_geometry(rois)
    out = _make_sc_pool()(table, idx.reshape(NW, BPW, K), vmul.reshape(NW, BPW, K))
    return out.reshape(R, CH, L, PH, PW)


# speedup vs baseline: 2.4516x; 2.4516x over previous
import functools

import jax
import jax.numpy as jnp
from jax import lax
from jax.experimental import pallas as pl
from jax.experimental.pallas import tpu as pltpu
from jax.experimental.pallas import tpu_sc as plsc

BS, CH, L, H, W = 2, 256, 8, 50, 50
R = 64
PH, PW = 7, 7
SCALE = 0.0625

D = CH * L
NPIX = BS * H * W
NBINS = R * PH * PW
NW = 32
BPW = NBINS // NW
K = 16
LANES = 16


def _bin_geometry(rois):
    b = jnp.clip(jnp.round(rois[:, 0]).astype(jnp.int32), 0, BS - 1)
    rsw = jnp.round(rois[:, 1] * SCALE).astype(jnp.int32)
    rsh = jnp.round(rois[:, 2] * SCALE).astype(jnp.int32)
    rew = jnp.round(rois[:, 3] * SCALE).astype(jnp.int32)
    reh = jnp.round(rois[:, 4] * SCALE).astype(jnp.int32)
    roi_w = jnp.maximum(rew - rsw + 1, 1)
    roi_h = jnp.maximum(reh - rsh + 1, 1)
    p = jnp.arange(PH, dtype=jnp.int32)
    hs = jnp.clip(p[None] * roi_h[:, None] // PH + rsh[:, None], 0, H)
    he = jnp.clip(((p[None] + 1) * roi_h[:, None] + PH - 1) // PH + rsh[:, None], 0, H)
    ws = jnp.clip(p[None] * roi_w[:, None] // PW + rsw[:, None], 0, W)
    we = jnp.clip(((p[None] + 1) * roi_w[:, None] + PW - 1) // PW + rsw[:, None], 0, W)
    valid = (he[:, :, None] > hs[:, :, None]) & (we[:, None, :] > ws[:, None, :])
    d4 = jnp.arange(4, dtype=jnp.int32)
    hh = jnp.clip(jnp.minimum(hs[:, :, None] + d4, he[:, :, None] - 1), 0, H - 1)
    ww = jnp.clip(jnp.minimum(ws[:, :, None] + d4, we[:, :, None] - 1), 0, W - 1)
    pid = (b[:, None, None, None, None] * (H * W)
           + hh[:, :, None, :, None] * W
           + ww[:, None, :, None, :])
    idx = pid.reshape(NBINS, K).astype(jnp.int32)
    vmul = jnp.broadcast_to(
        valid.reshape(NBINS, 1).astype(jnp.float32), (NBINS, K))
    return idx, vmul


@functools.cache
def _make_sc_pool():
    mesh = plsc.VectorSubcoreMesh(core_axis_name="c", subcore_axis_name="s")

    @functools.partial(
        pl.kernel,
        out_type=jax.ShapeDtypeStruct((NBINS, 1, D), jnp.float32),
        mesh=mesh,
        compiler_params=pltpu.CompilerParams(needs_layout_passes=False),
        scratch_types=[
            pltpu.VMEM((BPW, K), jnp.int32),
            pltpu.VMEM((BPW, K), jnp.float32),
            pltpu.VMEM((K, D), jnp.float32),
            pltpu.VMEM((1, D), jnp.float32),
            pltpu.SemaphoreType.DMA,
        ],
    )
    def _sc_pool(table_hbm, idx_hbm, vmul_hbm, out_hbm,
                 idx_v, vmul_v, rows_v, orow_v, gsem):
        wid = lax.axis_index("s") * 2 + lax.axis_index("c")
        base = wid * BPW
        pltpu.sync_copy(idx_hbm.at[wid], idx_v)
        pltpu.sync_copy(vmul_hbm.at[wid], vmul_v)

        @pl.loop(0, BPW)
        def _bin_loop(i):
            pltpu.async_copy(table_hbm.at[idx_v.at[i]], rows_v, gsem).wait()
            fvec = vmul_v[i, :]

            @pl.loop(0, D // LANES)
            def _d_loop(d):
                sl = pl.ds(d * LANES, LANES)
                acc = rows_v[0, sl]
                for k in range(1, K):
                    acc = jnp.maximum(acc, rows_v[k, sl])
                orow_v[0, sl] = acc * fvec

            pltpu.sync_copy(orow_v, out_hbm.at[base + i])

    return _sc_pool


def kernel(input, rois):
    table = jnp.transpose(input, (0, 3, 4, 2, 1)).reshape(NPIX, D)
    idx, vmul = _bin_geometry(rois)
    out = _make_sc_pool()(table, idx.reshape(NW, BPW, K), vmul.reshape(NW, BPW, K))
    return jnp.transpose(out.reshape(R, PH, PW, L, CH), (0, 4, 3, 1, 2))


# speedup vs baseline: 3.6258x; 1.4789x over previous
import functools

import jax
import jax.numpy as jnp
from jax import lax
from jax.experimental import pallas as pl
from jax.experimental.pallas import tpu as pltpu
from jax.experimental.pallas import tpu_sc as plsc

BS, CH, L, H, W = 2, 256, 8, 50, 50
R = 64
PH, PW = 7, 7
SCALE = 0.0625

D = CH * L
NPIX = BS * H * W
NBINS = R * PH * PW
NW = 32
BPW = NBINS // NW
K = 16
LANES = 16


def _bin_geometry(rois):
    b = jnp.clip(jnp.round(rois[:, 0]).astype(jnp.int32), 0, BS - 1)
    rsw = jnp.round(rois[:, 1] * SCALE).astype(jnp.int32)
    rsh = jnp.round(rois[:, 2] * SCALE).astype(jnp.int32)
    rew = jnp.round(rois[:, 3] * SCALE).astype(jnp.int32)
    reh = jnp.round(rois[:, 4] * SCALE).astype(jnp.int32)
    roi_w = jnp.maximum(rew - rsw + 1, 1)
    roi_h = jnp.maximum(reh - rsh + 1, 1)
    p = jnp.arange(PH, dtype=jnp.int32)
    hs = jnp.clip(p[None] * roi_h[:, None] // PH + rsh[:, None], 0, H)
    he = jnp.clip(((p[None] + 1) * roi_h[:, None] + PH - 1) // PH + rsh[:, None], 0, H)
    ws = jnp.clip(p[None] * roi_w[:, None] // PW + rsw[:, None], 0, W)
    we = jnp.clip(((p[None] + 1) * roi_w[:, None] + PW - 1) // PW + rsw[:, None], 0, W)
    valid = (he[:, :, None] > hs[:, :, None]) & (we[:, None, :] > ws[:, None, :])
    d4 = jnp.arange(4, dtype=jnp.int32)
    hh = jnp.clip(jnp.minimum(hs[:, :, None] + d4, he[:, :, None] - 1), 0, H - 1)
    ww = jnp.clip(jnp.minimum(ws[:, :, None] + d4, we[:, :, None] - 1), 0, W - 1)
    pid = (b[:, None, None, None, None] * (H * W)
           + hh[:, :, None, :, None] * W
           + ww[:, None, :, None, :])
    idx = pid.reshape(NBINS, K).astype(jnp.int32)
    vmul = jnp.broadcast_to(
        valid.reshape(NBINS, 1).astype(jnp.float32), (NBINS, K))
    return idx, vmul


@functools.cache
def _make_sc_pool():
    mesh = plsc.VectorSubcoreMesh(core_axis_name="c", subcore_axis_name="s")

    @functools.partial(
        pl.kernel,
        out_type=jax.ShapeDtypeStruct((NBINS, 1, D), jnp.float32),
        mesh=mesh,
        compiler_params=pltpu.CompilerParams(needs_layout_passes=False),
        scratch_types=[
            pltpu.VMEM((BPW, K), jnp.int32),
            pltpu.VMEM((BPW, K), jnp.float32),
            pltpu.VMEM((2, K, D), jnp.float32),
            pltpu.VMEM((1, D), jnp.float32),
            pltpu.SemaphoreType.DMA,
        ],
    )
    def _sc_pool(table_hbm, idx_hbm, vmul_hbm, out_hbm,
                 idx_v, vmul_v, rows_v, orow_v, gsem):
        wid = lax.axis_index("s") * 2 + lax.axis_index("c")
        base = wid * BPW
        pltpu.sync_copy(idx_hbm.at[wid], idx_v)
        pltpu.sync_copy(vmul_hbm.at[wid], vmul_v)
        pltpu.async_copy(table_hbm.at[idx_v.at[0]], rows_v.at[0], gsem)

        @pl.loop(0, BPW)
        def _bin_loop(i):
            p = lax.rem(i, 2)

            @pl.when(i + 1 < BPW)
            def _prefetch():
                pltpu.async_copy(
                    table_hbm.at[idx_v.at[i + 1]], rows_v.at[1 - p], gsem)

            pltpu.make_async_copy(
                table_hbm.at[idx_v.at[i]], rows_v.at[p], gsem).wait()
            fvec = vmul_v[i, :]

            @pl.loop(0, D // LANES)
            def _d_loop(d):
                sl = pl.ds(d * LANES, LANES)
                acc = rows_v[p, 0, sl]
                for k in range(1, K):
                    acc = jnp.maximum(acc, rows_v[p, k, sl])
                orow_v[0, sl] = acc * fvec

            pltpu.sync_copy(orow_v, out_hbm.at[base + i])

    return _sc_pool


def kernel(input, rois):
    table = jnp.transpose(input, (0, 3, 4, 2, 1)).reshape(NPIX, D)
    idx, vmul = _bin_geometry(rois)
    out = _make_sc_pool()(table, idx.reshape(NW, BPW, K), vmul.reshape(NW, BPW, K))
    return jnp.transpose(out.reshape(R, PH, PW, L, CH), (0, 4, 3, 1, 2))


# speedup vs baseline: 4.6230x; 1.2750x over previous
import functools

import jax
import jax.numpy as jnp
from jax import lax
from jax.experimental import pallas as pl
from jax.experimental.pallas import tpu as pltpu
from jax.experimental.pallas import tpu_sc as plsc

BS, CH, L, H, W = 2, 256, 8, 50, 50
R = 64
PH, PW = 7, 7
SCALE = 0.0625

D = CH * L
NPIX = BS * H * W
NBINS = R * PH * PW
NW = 32
BPW = NBINS // NW
K = 16
LANES = 16


def _bin_geometry(rois):
    b = jnp.clip(jnp.round(rois[:, 0]).astype(jnp.int32), 0, BS - 1)
    rsw = jnp.round(rois[:, 1] * SCALE).astype(jnp.int32)
    rsh = jnp.round(rois[:, 2] * SCALE).astype(jnp.int32)
    rew = jnp.round(rois[:, 3] * SCALE).astype(jnp.int32)
    reh = jnp.round(rois[:, 4] * SCALE).astype(jnp.int32)
    roi_w = jnp.maximum(rew - rsw + 1, 1)
    roi_h = jnp.maximum(reh - rsh + 1, 1)
    p = jnp.arange(PH, dtype=jnp.int32)
    hs = jnp.clip(p[None] * roi_h[:, None] // PH + rsh[:, None], 0, H)
    he = jnp.clip(((p[None] + 1) * roi_h[:, None] + PH - 1) // PH + rsh[:, None], 0, H)
    ws = jnp.clip(p[None] * roi_w[:, None] // PW + rsw[:, None], 0, W)
    we = jnp.clip(((p[None] + 1) * roi_w[:, None] + PW - 1) // PW + rsw[:, None], 0, W)
    valid = (he[:, :, None] > hs[:, :, None]) & (we[:, None, :] > ws[:, None, :])
    d4 = jnp.arange(4, dtype=jnp.int32)
    hh = jnp.clip(jnp.minimum(hs[:, :, None] + d4, he[:, :, None] - 1), 0, H - 1)
    ww = jnp.clip(jnp.minimum(ws[:, :, None] + d4, we[:, :, None] - 1), 0, W - 1)
    pid = (b[:, None, None, None, None] * (H * W)
           + hh[:, :, None, :, None] * W
           + ww[:, None, :, None, :])
    idx = pid.reshape(NBINS, K).astype(jnp.int32)
    vmul = jnp.broadcast_to(
        valid.reshape(NBINS, 1).astype(jnp.float32), (NBINS, K))
    return idx, vmul


@functools.cache
def _make_sc_pool():
    mesh = plsc.VectorSubcoreMesh(core_axis_name="c", subcore_axis_name="s")

    @functools.partial(
        pl.kernel,
        out_type=jax.ShapeDtypeStruct((NBINS, 1, D), jnp.float32),
        mesh=mesh,
        compiler_params=pltpu.CompilerParams(needs_layout_passes=False),
        scratch_types=[
            pltpu.VMEM((BPW, K), jnp.int32),
            pltpu.VMEM((BPW, K), jnp.float32),
            pltpu.VMEM((2, K, D), jnp.float32),
            pltpu.VMEM((1, D), jnp.float32),
            pltpu.SemaphoreType.DMA,
        ],
    )
    def _sc_pool(table_hbm, idx_hbm, vmul_hbm, out_hbm,
                 idx_v, vmul_v, rows_v, orow_v, gsem):
        wid = lax.axis_index("s") * 2 + lax.axis_index("c")
        base = wid * BPW
        pltpu.sync_copy(idx_hbm.at[wid], idx_v)
        pltpu.sync_copy(vmul_hbm.at[wid], vmul_v)
        pltpu.async_copy(table_hbm.at[idx_v.at[0]], rows_v.at[0], gsem)

        @pl.loop(0, BPW)
        def _bin_loop(i):
            p = lax.rem(i, 2)

            @pl.when(i + 1 < BPW)
            def _prefetch():
                pltpu.async_copy(
                    table_hbm.at[idx_v.at[i + 1]], rows_v.at[1 - p], gsem)

            pltpu.make_async_copy(
                table_hbm.at[idx_v.at[i]], rows_v.at[p], gsem).wait()
            fvec = vmul_v[i, :]

            @pl.loop(0, D // LANES)
            def _d_loop(d):
                sl = pl.ds(d * LANES, LANES)
                acc = rows_v[p, 0, sl]
                for k in range(1, K):
                    acc = jnp.maximum(acc, rows_v[p, k, sl])
                orow_v[0, sl] = acc * fvec

            pltpu.sync_copy(orow_v, out_hbm.at[base + i])

    return _sc_pool


def kernel(input, rois):
    table = (jnp.transpose(input, (0, 3, 4, 2, 1))
             .reshape(BS, H, W, L, CH // 128, 128)
             .transpose(0, 1, 2, 4, 3, 5)
             .reshape(NPIX, D))
    idx, vmul = _bin_geometry(rois)
    out = _make_sc_pool()(table, idx.reshape(NW, BPW, K), vmul.reshape(NW, BPW, K))
    out = out.reshape(R, PH, PW, CH // 128, L, 128)
    return jnp.transpose(out, (0, 3, 5, 4, 1, 2)).reshape(R, CH, L, PH, PW)


# speedup vs baseline: 5.4200x; 1.1724x over previous
import functools

import jax
import jax.numpy as jnp
from jax import lax
from jax.experimental import pallas as pl
from jax.experimental.pallas import tpu as pltpu
from jax.experimental.pallas import tpu_sc as plsc

BS, CH, L, H, W = 2, 256, 8, 50, 50
R = 64
PH, PW = 7, 7
SCALE = 0.0625

D = CH * L
NPIX = BS * H * W
NBINS = R * PH * PW
NW = 32
BPW = NBINS // NW
K = 16
LANES = 16


def _bin_geometry(rois):
    b = jnp.clip(jnp.round(rois[:, 0]).astype(jnp.int32), 0, BS - 1)
    rsw = jnp.round(rois[:, 1] * SCALE).astype(jnp.int32)
    rsh = jnp.round(rois[:, 2] * SCALE).astype(jnp.int32)
    rew = jnp.round(rois[:, 3] * SCALE).astype(jnp.int32)
    reh = jnp.round(rois[:, 4] * SCALE).astype(jnp.int32)
    roi_w = jnp.maximum(rew - rsw + 1, 1)
    roi_h = jnp.maximum(reh - rsh + 1, 1)
    p = jnp.arange(PH, dtype=jnp.int32)
    hs = jnp.clip(p[None] * roi_h[:, None] // PH + rsh[:, None], 0, H)
    he = jnp.clip(((p[None] + 1) * roi_h[:, None] + PH - 1) // PH + rsh[:, None], 0, H)
    ws = jnp.clip(p[None] * roi_w[:, None] // PW + rsw[:, None], 0, W)
    we = jnp.clip(((p[None] + 1) * roi_w[:, None] + PW - 1) // PW + rsw[:, None], 0, W)
    valid = (he[:, :, None] > hs[:, :, None]) & (we[:, None, :] > ws[:, None, :])
    d4 = jnp.arange(4, dtype=jnp.int32)
    hh = jnp.clip(jnp.minimum(hs[:, :, None] + d4, he[:, :, None] - 1), 0, H - 1)
    ww = jnp.clip(jnp.minimum(ws[:, :, None] + d4, we[:, :, None] - 1), 0, W - 1)
    pid = (b[:, None, None, None, None] * (H * W)
           + hh[:, :, None, :, None] * W
           + ww[:, None, :, None, :])
    idx = pid.reshape(NBINS, K).astype(jnp.int32)
    vmul = jnp.broadcast_to(
        valid.reshape(NBINS, 1).astype(jnp.float32), (NBINS, K))
    return idx, vmul


@functools.cache
def _make_sc_pool():
    mesh = plsc.VectorSubcoreMesh(core_axis_name="c", subcore_axis_name="s")

    @functools.partial(
        pl.kernel,
        out_type=jax.ShapeDtypeStruct((NBINS, 1, D), jnp.float32),
        mesh=mesh,
        compiler_params=pltpu.CompilerParams(
            needs_layout_passes=False, use_tc_tiling_on_sc=False),
        scratch_types=[
            pltpu.VMEM((BPW, K), jnp.int32),
            pltpu.VMEM((BPW, K), jnp.float32),
            pltpu.VMEM((2, K, D), jnp.float32),
            pltpu.VMEM((1, D), jnp.float32),
            pltpu.SemaphoreType.DMA,
        ],
    )
    def _sc_pool(table_hbm, idx_hbm, vmul_hbm, out_hbm,
                 idx_v, vmul_v, rows_v, orow_v, gsem):
        wid = lax.axis_index("s") * 2 + lax.axis_index("c")
        base = wid * BPW
        pltpu.sync_copy(idx_hbm.at[wid], idx_v)
        pltpu.sync_copy(vmul_hbm.at[wid], vmul_v)
        pltpu.async_copy(table_hbm.at[idx_v.at[0]], rows_v.at[0], gsem)

        @pl.loop(0, BPW)
        def _bin_loop(i):
            p = lax.rem(i, 2)

            @pl.when(i + 1 < BPW)
            def _prefetch():
                pltpu.async_copy(
                    table_hbm.at[idx_v.at[i + 1]], rows_v.at[1 - p], gsem)

            pltpu.make_async_copy(
                table_hbm.at[idx_v.at[i]], rows_v.at[p], gsem).wait()
            fvec = vmul_v[i, :]

            @pl.loop(0, D // LANES)
            def _d_loop(d):
                sl = pl.ds(d * LANES, LANES)
                acc = rows_v[p, 0, sl]
                for k in range(1, K):
                    acc = jnp.maximum(acc, rows_v[p, k, sl])
                orow_v[0, sl] = acc * fvec

            pltpu.sync_copy(orow_v, out_hbm.at[base + i])

    return _sc_pool


def kernel(input, rois):
    table = (jnp.transpose(input, (0, 3, 4, 2, 1))
             .reshape(BS, H, W, L, CH // 128, 128)
             .transpose(0, 1, 2, 4, 3, 5)
             .reshape(NPIX, D))
    idx, vmul = _bin_geometry(rois)
    out = _make_sc_pool()(table, idx.reshape(NW, BPW, K), vmul.reshape(NW, BPW, K))
    out = out.reshape(R, PH, PW, CH // 128, L, 128)
    return jnp.transpose(out, (0, 3, 5, 4, 1, 2)).reshape(R, CH, L, PH, PW)
